# baseline (speedup 1.0000x reference)
import jax
import jax.numpy as jnp
from jax import lax
from jax.experimental import pallas as pl
from jax.experimental.pallas import tpu as pltpu
from jax.experimental.pallas import tpu_sc as plsc

D_MODEL = 256
NTOK = 16 * 2048
DM = D_MODEL // 2
NHR = NTOK * 2

NC, NS, LANES = 2, 16, 16
NW = NC * NS
TPW = NTOK // NW
CHUNK = 64
NCHUNK = TPW // CHUNK
SLOTS = 2 * CHUNK
NBUF = 3


def _sc_body(x128, cf, tab, out, coordv, *bufs):
    sets = [bufs[i * 5:(i + 1) * 5] for i in range(NBUF)]
    wid = lax.axis_index("s") * NC + lax.axis_index("c")
    tok0 = wid * TPW
    pltpu.sync_copy(cf.at[pl.ds(tok0 * 2, TPW * 2)], coordv)
    lane = lax.iota(jnp.int32, LANES)
    swapped = lane ^ 1

    def issue_in(c, S):
        idxv, xv, sem_x, _, _ = S
        hx = pltpu.async_copy(
            x128.at[pl.ds((tok0 + c * CHUNK) * 2, SLOTS), :], xv, sem_x)
        cbase = c * SLOTS
        for g in range(SLOTS // LANES):
            v0 = coordv[pl.ds(cbase + g * LANES, LANES)]
            v = lax.gather(
                v0, swapped[:, None],
                dimension_numbers=lax.GatherDimensionNumbers(
                    offset_dims=(), collapsed_slice_dims=(0,),
                    start_index_map=(0,)),
                slice_sizes=(1,),
                mode=lax.GatherScatterMode.PROMISE_IN_BOUNDS)
            idxv[pl.ds(g * LANES, LANES)] = (v / 100.0).astype(jnp.int32)
        return hx

    hx, hg, hout = {}, {}, {}
    for t in range(NCHUNK + 2):
        cA, cB, cC = t, t - 1, t - 2
        if cA < NCHUNK:
            S = sets[cA % NBUF]
            if cA >= NBUF:
                hout.pop(cA - NBUF).wait()
            hx[cA] = issue_in(cA, S)
        if 0 <= cB < NCHUNK:
            S = sets[cB % NBUF]
            hx.pop(cB).wait()
            hg[cB] = pltpu.async_copy(tab.at[S[0]], S[1], S[3], add=True)
        if 0 <= cC < NCHUNK:
            S = sets[cC % NBUF]
            hg.pop(cC).wait()
            hout[cC] = pltpu.async_copy(
                S[1], out.at[pl.ds((tok0 + cC * CHUNK) * 2, SLOTS), :], S[4])
    for c in sorted(hout):
        hout.pop(c).wait()


def _buf_set():
    return [
        pltpu.VMEM((SLOTS,), jnp.int32),
        pltpu.VMEM((SLOTS, DM), jnp.float32),
        pltpu.SemaphoreType.DMA,
        pltpu.SemaphoreType.DMA,
        pltpu.SemaphoreType.DMA,
    ]


_sc_call = pl.kernel(
    _sc_body,
    out_type=jax.ShapeDtypeStruct((NHR, DM), jnp.float32),
    mesh=plsc.VectorSubcoreMesh(
        core_axis_name="c", subcore_axis_name="s",
        num_cores=NC, num_subcores=NS,
    ),
    scratch_types=[pltpu.VMEM((TPW * 2,), jnp.float32)]
    + _buf_set() + _buf_set() + _buf_set(),
)


@jax.jit
def kernel(x, coord, pe):
    tab = (0.1 * pe[:DM, 0, :]).T
    out2 = _sc_call(x.reshape(NHR, DM), coord.reshape(-1), tab)
    return out2.reshape(x.shape)


# speedup vs baseline: 1.5307x; 1.5307x over previous
import jax
import jax.numpy as jnp
from jax import lax
from jax.experimental import pallas as pl
from jax.experimental.pallas import tpu as pltpu
from jax.experimental.pallas import tpu_sc as plsc

D_MODEL = 256
NTOK = 16 * 2048
DM = D_MODEL // 2

NC, NS, LANES = 2, 16, 16
NW = NC * NS
TPW = NTOK // NW
CHUNK = 64
NCHUNK = TPW // CHUNK
SLOTS = 2 * CHUNK
NBUF = 2


def _tab_body(src, out):
    out[...] = 0.1 * src[...].T


_tab_call = pl.pallas_call(
    _tab_body,
    out_shape=jax.ShapeDtypeStruct((384, DM), jnp.float32),
)


def _sc_body(x2, cf, tab, out, coordv, *bufs):
    sets = [bufs[i * 6:(i + 1) * 6] for i in range(NBUF)]
    wid = lax.axis_index("s") * NC + lax.axis_index("c")
    tok0 = wid * TPW
    pltpu.sync_copy(cf.at[pl.ds(tok0 * 2, TPW * 2)], coordv)

    def issue_in(c, S):
        idxv, xv, rowsv, sem_x, sem_g, _ = S
        cbase = c * SLOTS
        for g in range(SLOTS // LANES):
            v = coordv[pl.ds(cbase + g * LANES, LANES)]
            idxv[pl.ds(g * LANES, LANES)] = (v / 100.0).astype(jnp.int32)
        hx = pltpu.async_copy(x2.at[pl.ds(tok0 + c * CHUNK, CHUNK), :], xv, sem_x)
        hg = pltpu.async_copy(tab.at[idxv], rowsv, sem_g)
        return hx, hg

    def accumulate(S):
        idxv, xv, rowsv, *_ = S

        def add_body(s, acc):
            cb = (1 - (s & 1)) * DM
            for k in range(DM // LANES):
                v = rowsv[s, pl.ds(k * LANES, LANES)]
                plsc.addupdate(xv.at[s >> 1, pl.ds(cb + k * LANES, LANES)], v)
            return acc

        lax.fori_loop(0, SLOTS, add_body, 0)

    inflight = {}
    pending_out = {}
    for c in range(NCHUNK):
        if c == 0:
            inflight[0] = issue_in(0, sets[0])
        if c + 1 < NCHUNK:
            if c >= 1:
                pending_out.pop(c - 1).wait()
            inflight[c + 1] = issue_in(c + 1, sets[(c + 1) % NBUF])
        hx, hg = inflight.pop(c)
        hx.wait()
        hg.wait()
        S = sets[c % NBUF]
        accumulate(S)
        pending_out[c] = pltpu.async_copy(
            S[1], out.at[pl.ds(tok0 + c * CHUNK, CHUNK), :], S[5])
    for c in sorted(pending_out):
        pending_out.pop(c).wait()


def _buf_set():
    return [
        pltpu.VMEM((SLOTS,), jnp.int32),
        pltpu.VMEM((CHUNK, D_MODEL), jnp.float32),
        pltpu.VMEM((SLOTS, DM), jnp.float32),
        pltpu.SemaphoreType.DMA,
        pltpu.SemaphoreType.DMA,
        pltpu.SemaphoreType.DMA,
    ]


_sc_call = pl.kernel(
    _sc_body,
    out_type=jax.ShapeDtypeStruct((NTOK, D_MODEL), jnp.float32),
    mesh=plsc.VectorSubcoreMesh(
        core_axis_name="c", subcore_axis_name="s",
        num_cores=NC, num_subcores=NS,
    ),
    scratch_types=[pltpu.VMEM((TPW * 2,), jnp.float32)]
    + _buf_set() + _buf_set(),
)


@jax.jit
def kernel(x, coord, pe):
    tab = _tab_call(pe[:DM, 0, :])
    out2 = _sc_call(x.reshape(NTOK, D_MODEL), coord.reshape(-1), tab)
    return out2.reshape(x.shape)
